# baseline (speedup 1.0000x reference)
import functools
import jax
import jax.numpy as jnp
from jax import lax
from jax.experimental import pallas as pl
from jax.experimental.pallas import tpu as pltpu
from jax.experimental.pallas import tpu_sc as plsc

TH = 0.5
L = 16
NWORK = 32
CH = 64
RB = 16


def _treemax(vs):
    t = list(vs)
    while len(t) > 1:
        nxt = [jnp.maximum(t[2 * i], t[2 * i + 1]) for i in range(len(t) // 2)]
        if len(t) % 2:
            nxt.append(t[-1])
        t = nxt
    return t[0]


def _sc_bbox(N, H, W):
    spw = N // NWORK
    nch = H // CH
    gch = N * nch
    cpw = spw * nch
    niter = cpw // 2

    def body(m_hbm, out_hbm, buf0, buf1, colm, obuf, sem0, sem1):
        wid = lax.axis_index("s") * 2 + lax.axis_index("c")
        base = wid * cpw
        lanes = lax.iota(jnp.int32, L)
        neg = jnp.full((L,), -jnp.inf, jnp.float32)
        bufs = (buf0, buf1)
        sems = (sem0, sem1)

        def start(gi, b):
            pltpu.async_copy(m_hbm.at[gi], bufs[b], sems[b])

        def wait(b):
            pltpu.make_async_copy(m_hbm.at[0], bufs[b], sems[b]).wait()

        def init_colm():
            def ib(g, _):
                colm[pl.ds(g * L, L)] = neg
                return 0
            lax.fori_loop(0, W // L, ib, 0)

        def chunk_reduce(b, lc, ymin, ymax):
            buf = bufs[b]

            def rowblock(rb, carry):
                ymin, ymax = carry

                def gloop(g, rowps):
                    vs = [buf[rb * RB + i, pl.ds(g * L, L)] for i in range(RB)]
                    rowps = tuple(jnp.maximum(rowps[i], vs[i]) for i in range(RB))
                    colm[pl.ds(g * L, L)] = jnp.maximum(colm[pl.ds(g * L, L)], _treemax(vs))
                    return rowps

                rowps = lax.fori_loop(
                    0, W // L, gloop,
                    tuple(jnp.full((L,), -jnp.inf, jnp.float32) for _ in range(RB)))
                hbase = lc * CH + rb * RB
                for i in range(RB):
                    occ = plsc.all_reduce_population_count(rowps[i] >= TH) > 0
                    h = jnp.full((L,), hbase + i, jnp.int32)
                    ymin = jnp.where(occ, jnp.minimum(ymin, h), ymin)
                    ymax = jnp.where(occ, jnp.maximum(ymax, h), ymax)
                return ymin, ymax

            return lax.fori_loop(0, CH // RB, rowblock, (ymin, ymax))

        def extract_x():
            def xloop(g, carry):
                xminv, xmaxv = carry
                v = colm[pl.ds(g * L, L)]
                occ = v >= TH
                has = plsc.all_reduce_population_count(occ) > 0
                first = plsc.all_reduce_ffs(occ)
                last = (L - 1) - plsc.all_reduce_ffs(lax.rev(v, (0,)) >= TH)
                xminv = jnp.where(has, jnp.minimum(xminv, g * L + first), xminv)
                xmaxv = jnp.where(has, jnp.maximum(xmaxv, g * L + last), xmaxv)
                return xminv, xmaxv

            return lax.fori_loop(
                0, W // L, xloop,
                (jnp.full((L,), W, jnp.int32), jnp.full((L,), -1, jnp.int32)))

        init_colm()
        start(base, 0)

        def ring_iter(i, carry):
            ymin, ymax, outv0, outv1 = carry
            g0 = base + 2 * i
            g1 = g0 + 1
            lc0 = lax.rem(2 * i, nch)
            lc1 = lax.rem(2 * i + 1, nch)

            start(g1, 1)
            wait(0)
            ymin, ymax = chunk_reduce(0, lc0, ymin, ymax)
            start(jnp.minimum(g0 + 2, gch - 1), 0)
            wait(1)
            ymin, ymax = chunk_reduce(1, lc1, ymin, ymax)

            done = lc1 == (nch - 1)
            xmin, xmax = extract_x()
            any_w = xmax >= 0
            any_h = ymax >= 0
            vals = (
                jnp.where(any_h, ymin, 0),
                jnp.where(any_w, xmin, 0),
                jnp.where(any_h, ymax + 1, H),
                jnp.where(any_w, xmax + 1, W),
            )
            smp = (2 * i + 1) // nch
            vi = smp // 4
            lane0 = lax.rem(smp, 4) * 4
            for j, val in enumerate(vals):
                hit = done & (lanes == (lane0 + j))
                outv0 = jnp.where(hit & (vi == 0), val, outv0)
                outv1 = jnp.where(hit & (vi == 1), val, outv1)

            @pl.when(done)
            def _():
                init_colm()

            ymin = jnp.where(done, jnp.full((L,), H, jnp.int32), ymin)
            ymax = jnp.where(done, jnp.full((L,), -1, jnp.int32), ymax)
            return ymin, ymax, outv0, outv1

        z = jnp.zeros((L,), jnp.int32)
        _, _, outv0, outv1 = lax.fori_loop(
            0, niter, ring_iter,
            (jnp.full((L,), H, jnp.int32), jnp.full((L,), -1, jnp.int32), z, z))
        wait(0)

        obuf[pl.ds(0, L)] = outv0
        obuf[pl.ds(L, L)] = outv1
        pltpu.sync_copy(obuf, out_hbm.at[pl.ds(wid * 2 * L, 2 * L)])

    mesh = plsc.VectorSubcoreMesh(core_axis_name="c", subcore_axis_name="s")
    return pl.kernel(
        body,
        out_type=jax.ShapeDtypeStruct((N * 4,), jnp.int32),
        mesh=mesh,
        compiler_params=pltpu.CompilerParams(needs_layout_passes=False),
        scratch_types=[
            pltpu.VMEM((CH, W), jnp.float32),
            pltpu.VMEM((CH, W), jnp.float32),
            pltpu.VMEM((W,), jnp.float32),
            pltpu.VMEM((2 * L,), jnp.int32),
            pltpu.SemaphoreType.DMA,
            pltpu.SemaphoreType.DMA,
        ],
    )


BLK_N = 8


def _tc_body(m_ref, out_ref):
    m = m_ref[...]
    _, H, W = m.shape
    colmax = jnp.max(m, axis=1)
    rowmax = jnp.max(m, axis=2)
    wocc = colmax >= TH
    hocc = rowmax >= TH
    iw = lax.broadcasted_iota(jnp.int32, wocc.shape, 1)
    ih = lax.broadcasted_iota(jnp.int32, hocc.shape, 1)
    any_w = jnp.any(wocc, axis=1)
    any_h = jnp.any(hocc, axis=1)
    xmin = jnp.where(any_w, jnp.min(jnp.where(wocc, iw, W), axis=1), 0)
    xmax = jnp.where(any_w, jnp.max(jnp.where(wocc, iw, -1), axis=1) + 1, W)
    ymin = jnp.where(any_h, jnp.min(jnp.where(hocc, ih, H), axis=1), 0)
    ymax = jnp.where(any_h, jnp.max(jnp.where(hocc, ih, -1), axis=1) + 1, H)
    out_ref[...] = jnp.stack((ymin, xmin, ymax, xmax), axis=-1)


def _tc_bbox(m):
    N, H, W = m.shape
    return pl.pallas_call(
        _tc_body,
        grid=(N // BLK_N,),
        in_specs=[pl.BlockSpec((BLK_N, H, W), lambda i: (i, 0, 0))],
        out_specs=pl.BlockSpec((BLK_N, 4), lambda i: (i, 0)),
        out_shape=jax.ShapeDtypeStruct((N, 4), jnp.int32),
    )(m)


N_SC = 96


def kernel(mask):
    N, _, H, W = mask.shape
    m = mask.reshape(N, H, W)
    m_sc = m[:N_SC].reshape(N_SC * (H // CH), CH, W)
    out_sc = _sc_bbox(N_SC, H, W)(m_sc)
    out_tc = _tc_bbox(m[N_SC:])
    return jnp.concatenate((out_sc.reshape(N_SC, 4), out_tc), axis=0)


# speedup vs baseline: 2.6288x; 2.6288x over previous
import functools
import jax
import jax.numpy as jnp
from jax import lax
from jax.experimental import pallas as pl
from jax.experimental.pallas import tpu as pltpu
from jax.experimental.pallas import tpu_sc as plsc

TH = 0.5
L = 16
NWORK = 32
CH = 64
RB = 16


def _treemax(vs):
    t = list(vs)
    while len(t) > 1:
        nxt = [jnp.maximum(t[2 * i], t[2 * i + 1]) for i in range(len(t) // 2)]
        if len(t) % 2:
            nxt.append(t[-1])
        t = nxt
    return t[0]


def _sc_bbox(N, n_sc, H, W):
    spw = n_sc // NWORK
    nch = H // CH
    gch = n_sc * nch
    cpw = spw * nch
    niter = cpw // 2

    def body(m_hbm, out_hbm, buf0, buf1, colm, obuf, sem0, sem1):
        wid = lax.axis_index("s") * 2 + lax.axis_index("c")
        base = wid * cpw
        lanes = lax.iota(jnp.int32, L)
        neg = jnp.full((L,), -jnp.inf, jnp.float32)
        bufs = (buf0, buf1)
        sems = (sem0, sem1)

        def start(gi, b):
            pltpu.async_copy(m_hbm.at[gi], bufs[b], sems[b])

        def wait(b):
            pltpu.make_async_copy(m_hbm.at[0], bufs[b], sems[b]).wait()

        def init_colm():
            def ib(g, _):
                colm[pl.ds(g * L, L)] = neg
                return 0
            lax.fori_loop(0, W // L, ib, 0)

        def chunk_reduce(b, lc, ymin, ymax):
            buf = bufs[b]

            def rowblock(rb, carry):
                ymin, ymax = carry

                def gloop(g, rowps):
                    vs = [buf[rb * RB + i, pl.ds(g * L, L)] for i in range(RB)]
                    rowps = tuple(jnp.maximum(rowps[i], vs[i]) for i in range(RB))
                    colm[pl.ds(g * L, L)] = jnp.maximum(colm[pl.ds(g * L, L)], _treemax(vs))
                    return rowps

                rowps = lax.fori_loop(
                    0, W // L, gloop,
                    tuple(jnp.full((L,), -jnp.inf, jnp.float32) for _ in range(RB)))
                hbase = lc * CH + rb * RB
                for i in range(RB):
                    occ = plsc.all_reduce_population_count(rowps[i] >= TH) > 0
                    h = jnp.full((L,), hbase + i, jnp.int32)
                    ymin = jnp.where(occ, jnp.minimum(ymin, h), ymin)
                    ymax = jnp.where(occ, jnp.maximum(ymax, h), ymax)
                return ymin, ymax

            return lax.fori_loop(0, CH // RB, rowblock, (ymin, ymax))

        def extract_x():
            def xloop(g, carry):
                xminv, xmaxv = carry
                v = colm[pl.ds(g * L, L)]
                occ = v >= TH
                has = plsc.all_reduce_population_count(occ) > 0
                first = plsc.all_reduce_ffs(occ)
                last = (L - 1) - plsc.all_reduce_ffs(lax.rev(v, (0,)) >= TH)
                xminv = jnp.where(has, jnp.minimum(xminv, g * L + first), xminv)
                xmaxv = jnp.where(has, jnp.maximum(xmaxv, g * L + last), xmaxv)
                return xminv, xmaxv

            return lax.fori_loop(
                0, W // L, xloop,
                (jnp.full((L,), W, jnp.int32), jnp.full((L,), -1, jnp.int32)))

        init_colm()
        start(base, 0)

        def ring_iter(i, carry):
            ymin, ymax, outv0, outv1 = carry
            g0 = base + 2 * i
            g1 = g0 + 1
            lc0 = lax.rem(2 * i, nch)
            lc1 = lax.rem(2 * i + 1, nch)

            start(g1, 1)
            wait(0)
            ymin, ymax = chunk_reduce(0, lc0, ymin, ymax)
            start(jnp.minimum(g0 + 2, gch - 1), 0)
            wait(1)
            ymin, ymax = chunk_reduce(1, lc1, ymin, ymax)

            done = lc1 == (nch - 1)
            xmin, xmax = extract_x()
            any_w = xmax >= 0
            any_h = ymax >= 0
            vals = (
                jnp.where(any_h, ymin, 0),
                jnp.where(any_w, xmin, 0),
                jnp.where(any_h, ymax + 1, H),
                jnp.where(any_w, xmax + 1, W),
            )
            smp = (2 * i + 1) // nch
            vi = smp // 4
            lane0 = lax.rem(smp, 4) * 4
            for j, val in enumerate(vals):
                hit = done & (lanes == (lane0 + j))
                outv0 = jnp.where(hit & (vi == 0), val, outv0)
                outv1 = jnp.where(hit & (vi == 1), val, outv1)

            @pl.when(done)
            def _():
                init_colm()

            ymin = jnp.where(done, jnp.full((L,), H, jnp.int32), ymin)
            ymax = jnp.where(done, jnp.full((L,), -1, jnp.int32), ymax)
            return ymin, ymax, outv0, outv1

        z = jnp.zeros((L,), jnp.int32)
        _, _, outv0, outv1 = lax.fori_loop(
            0, niter, ring_iter,
            (jnp.full((L,), H, jnp.int32), jnp.full((L,), -1, jnp.int32), z, z))
        wait(0)

        obuf[pl.ds(0, L)] = outv0
        obuf[pl.ds(L, L)] = outv1
        pltpu.sync_copy(obuf, out_hbm.at[pl.ds(wid * 2 * L, 2 * L)])

    mesh = plsc.VectorSubcoreMesh(core_axis_name="c", subcore_axis_name="s")
    return pl.kernel(
        body,
        out_type=jax.ShapeDtypeStruct((NWORK * 2 * L,), jnp.int32),
        mesh=mesh,
        compiler_params=pltpu.CompilerParams(needs_layout_passes=False),
        scratch_types=[
            pltpu.VMEM((CH, W), jnp.float32),
            pltpu.VMEM((CH, W), jnp.float32),
            pltpu.VMEM((W,), jnp.float32),
            pltpu.VMEM((2 * L,), jnp.int32),
            pltpu.SemaphoreType.DMA,
            pltpu.SemaphoreType.DMA,
        ],
    )


BLK_N = 8


def _tc_body(m_ref, out_ref):
    m = m_ref[...]
    _, H, W = m.shape
    colmax = jnp.max(m, axis=1)
    rowmax = jnp.max(m, axis=2)
    wocc = colmax >= TH
    hocc = rowmax >= TH
    iw = lax.broadcasted_iota(jnp.int32, wocc.shape, 1)
    ih = lax.broadcasted_iota(jnp.int32, hocc.shape, 1)
    any_w = jnp.any(wocc, axis=1)
    any_h = jnp.any(hocc, axis=1)
    xmin = jnp.where(any_w, jnp.min(jnp.where(wocc, iw, W), axis=1), 0)
    xmax = jnp.where(any_w, jnp.max(jnp.where(wocc, iw, -1), axis=1) + 1, W)
    ymin = jnp.where(any_h, jnp.min(jnp.where(hocc, ih, H), axis=1), 0)
    ymax = jnp.where(any_h, jnp.max(jnp.where(hocc, ih, -1), axis=1) + 1, H)
    out_ref[...] = jnp.stack((ymin, xmin, ymax, xmax), axis=-1)


def _tc_bbox(m, n_skip):
    N, H, W = m.shape
    n_tc = N - n_skip
    blk0 = n_skip // BLK_N
    return pl.pallas_call(
        _tc_body,
        grid=(n_tc // BLK_N,),
        in_specs=[pl.BlockSpec((BLK_N, H, W), lambda i: (i + blk0, 0, 0))],
        out_specs=pl.BlockSpec((BLK_N, 4), lambda i: (i, 0)),
        out_shape=jax.ShapeDtypeStruct((n_tc, 4), jnp.int32),
    )(m)


N_SC = 96


def kernel(mask):
    N, _, H, W = mask.shape
    m = mask.reshape(N, H, W)
    m_chunks = mask.reshape(N * (H // CH), CH, W)
    spw = N_SC // NWORK
    out_sc = _sc_bbox(N, N_SC, H, W)(m_chunks)
    out_tc = _tc_bbox(m, N_SC)
    boxes_sc = out_sc.reshape(NWORK, 2 * L)[:, : spw * 4].reshape(N_SC, 4)
    return jnp.concatenate((boxes_sc, out_tc), axis=0)


# speedup vs baseline: 2.6303x; 1.0006x over previous
import functools
import jax
import jax.numpy as jnp
from jax import lax
from jax.experimental import pallas as pl
from jax.experimental.pallas import tpu as pltpu
from jax.experimental.pallas import tpu_sc as plsc

TH = 0.5
L = 16
NWORK = 32
CH = 64
RB = 16


def _treemax(vs):
    t = list(vs)
    while len(t) > 1:
        nxt = [jnp.maximum(t[2 * i], t[2 * i + 1]) for i in range(len(t) // 2)]
        if len(t) % 2:
            nxt.append(t[-1])
        t = nxt
    return t[0]


def _sc_bbox(N, n_sc, H, W):
    spw = n_sc // NWORK
    nch = H // CH
    gch = n_sc * nch
    cpw = spw * nch
    niter = cpw // 2

    def body(m_hbm, out_hbm, buf0, buf1, colm, obuf, sem0, sem1):
        wid = lax.axis_index("s") * 2 + lax.axis_index("c")
        base = wid * cpw
        lanes = lax.iota(jnp.int32, L)
        neg = jnp.full((L,), -jnp.inf, jnp.float32)
        bufs = (buf0, buf1)
        sems = (sem0, sem1)

        def start(gi, b):
            pltpu.async_copy(m_hbm.at[gi], bufs[b], sems[b])

        def wait(b):
            pltpu.make_async_copy(m_hbm.at[0], bufs[b], sems[b]).wait()

        def init_colm():
            def ib(g, _):
                colm[pl.ds(g * L, L)] = neg
                return 0
            lax.fori_loop(0, W // L, ib, 0)

        def chunk_reduce(b, lc, ymin, ymax):
            buf = bufs[b]

            def rowblock(rb, carry):
                ymin, ymax = carry

                def gloop(g, rowps):
                    vs = [buf[rb * RB + i, pl.ds(g * L, L)] for i in range(RB)]
                    rowps = tuple(jnp.maximum(rowps[i], vs[i]) for i in range(RB))
                    colm[pl.ds(g * L, L)] = jnp.maximum(colm[pl.ds(g * L, L)], _treemax(vs))
                    return rowps

                rowps = lax.fori_loop(
                    0, W // L, gloop,
                    tuple(jnp.full((L,), -jnp.inf, jnp.float32) for _ in range(RB)))
                hbase = lc * CH + rb * RB
                for i in range(RB):
                    occ = plsc.all_reduce_population_count(rowps[i] >= TH) > 0
                    h = jnp.full((L,), hbase + i, jnp.int32)
                    ymin = jnp.where(occ, jnp.minimum(ymin, h), ymin)
                    ymax = jnp.where(occ, jnp.maximum(ymax, h), ymax)
                return ymin, ymax

            return lax.fori_loop(0, CH // RB, rowblock, (ymin, ymax))

        def extract_x():
            def xloop(g, carry):
                xminv, xmaxv = carry
                v = colm[pl.ds(g * L, L)]
                occ = v >= TH
                has = plsc.all_reduce_population_count(occ) > 0
                first = plsc.all_reduce_ffs(occ)
                last = (L - 1) - plsc.all_reduce_ffs(lax.rev(v, (0,)) >= TH)
                xminv = jnp.where(has, jnp.minimum(xminv, g * L + first), xminv)
                xmaxv = jnp.where(has, jnp.maximum(xmaxv, g * L + last), xmaxv)
                return xminv, xmaxv

            return lax.fori_loop(
                0, W // L, xloop,
                (jnp.full((L,), W, jnp.int32), jnp.full((L,), -1, jnp.int32)))

        init_colm()
        start(base, 0)

        def ring_iter(i, carry):
            ymin, ymax, outv0, outv1 = carry
            g0 = base + 2 * i
            g1 = g0 + 1
            lc0 = lax.rem(2 * i, nch)
            lc1 = lax.rem(2 * i + 1, nch)

            start(g1, 1)
            wait(0)
            ymin, ymax = chunk_reduce(0, lc0, ymin, ymax)
            start(jnp.minimum(g0 + 2, gch - 1), 0)
            wait(1)
            ymin, ymax = chunk_reduce(1, lc1, ymin, ymax)

            done = lc1 == (nch - 1)
            xmin, xmax = extract_x()
            any_w = xmax >= 0
            any_h = ymax >= 0
            vals = (
                jnp.where(any_h, ymin, 0),
                jnp.where(any_w, xmin, 0),
                jnp.where(any_h, ymax + 1, H),
                jnp.where(any_w, xmax + 1, W),
            )
            smp = (2 * i + 1) // nch
            vi = smp // 4
            lane0 = lax.rem(smp, 4) * 4
            for j, val in enumerate(vals):
                hit = done & (lanes == (lane0 + j))
                outv0 = jnp.where(hit & (vi == 0), val, outv0)
                outv1 = jnp.where(hit & (vi == 1), val, outv1)

            @pl.when(done)
            def _():
                init_colm()

            ymin = jnp.where(done, jnp.full((L,), H, jnp.int32), ymin)
            ymax = jnp.where(done, jnp.full((L,), -1, jnp.int32), ymax)
            return ymin, ymax, outv0, outv1

        z = jnp.zeros((L,), jnp.int32)
        _, _, outv0, outv1 = lax.fori_loop(
            0, niter, ring_iter,
            (jnp.full((L,), H, jnp.int32), jnp.full((L,), -1, jnp.int32), z, z))
        wait(0)

        obuf[pl.ds(0, L)] = outv0
        obuf[pl.ds(L, L)] = outv1
        pltpu.sync_copy(obuf, out_hbm.at[pl.ds(wid * 2 * L, 2 * L)])

    mesh = plsc.VectorSubcoreMesh(core_axis_name="c", subcore_axis_name="s")
    return pl.kernel(
        body,
        out_type=jax.ShapeDtypeStruct((NWORK * 2 * L,), jnp.int32),
        mesh=mesh,
        compiler_params=pltpu.CompilerParams(needs_layout_passes=False),
        scratch_types=[
            pltpu.VMEM((CH, W), jnp.float32),
            pltpu.VMEM((CH, W), jnp.float32),
            pltpu.VMEM((W,), jnp.float32),
            pltpu.VMEM((2 * L,), jnp.int32),
            pltpu.SemaphoreType.DMA,
            pltpu.SemaphoreType.DMA,
        ],
    )


BLK_N = 16


def _tc_body(m_ref, out_ref):
    m = m_ref[...]
    _, H, W = m.shape
    colmax = jnp.max(m, axis=1)
    rowmax = jnp.max(m, axis=2)
    wocc = colmax >= TH
    hocc = rowmax >= TH
    iw = lax.broadcasted_iota(jnp.int32, wocc.shape, 1)
    ih = lax.broadcasted_iota(jnp.int32, hocc.shape, 1)
    any_w = jnp.any(wocc, axis=1)
    any_h = jnp.any(hocc, axis=1)
    xmin = jnp.where(any_w, jnp.min(jnp.where(wocc, iw, W), axis=1), 0)
    xmax = jnp.where(any_w, jnp.max(jnp.where(wocc, iw, -1), axis=1) + 1, W)
    ymin = jnp.where(any_h, jnp.min(jnp.where(hocc, ih, H), axis=1), 0)
    ymax = jnp.where(any_h, jnp.max(jnp.where(hocc, ih, -1), axis=1) + 1, H)
    out_ref[...] = jnp.stack((ymin, xmin, ymax, xmax), axis=-1)


def _tc_bbox(m, n_skip):
    N, H, W = m.shape
    n_tc = N - n_skip
    blk0 = n_skip // BLK_N
    return pl.pallas_call(
        _tc_body,
        grid=(n_tc // BLK_N,),
        in_specs=[pl.BlockSpec((BLK_N, H, W), lambda i: (i + blk0, 0, 0))],
        out_specs=pl.BlockSpec((BLK_N, 4), lambda i: (i, 0)),
        out_shape=jax.ShapeDtypeStruct((n_tc, 4), jnp.int32),
    )(m)


N_SC = 96


def kernel(mask):
    N, _, H, W = mask.shape
    m = mask.reshape(N, H, W)
    m_chunks = mask.reshape(N * (H // CH), CH, W)
    spw = N_SC // NWORK
    out_sc = _sc_bbox(N, N_SC, H, W)(m_chunks)
    out_tc = _tc_bbox(m, N_SC)
    boxes_sc = out_sc.reshape(NWORK, 2 * L)[:, : spw * 4].reshape(N_SC, 4)
    return jnp.concatenate((boxes_sc, out_tc), axis=0)
